# Initial kernel scaffold; baseline (speedup 1.0000x reference)
import jax
import jax.numpy as jnp
from jax.experimental import pallas as pl


def kernel(x, edge_index, idx, W1, b1, W2, b2):
    raise NotImplementedError("write your pallas kernel here")


# speedup vs baseline: 4.5710x; 4.5710x over previous
import functools

import jax
import jax.numpy as jnp
from jax import lax
from jax.experimental import pallas as pl
from jax.experimental.pallas import tpu as pltpu
from jax.experimental.pallas import tpu_sc as plsc

N = 10000
NP = 10240
E = 320000
CH = 128
NCH = 2560
EP = NCH * CH
NTILE = 16
RPT = NP // NTILE
HALF = NP // 2
NTR = NTILE
NPH = HALF + NTR
RPH = HALF // NTILE
CPT = NCH // NTILE

_MESH = dict(core_axis_name="c", subcore_axis_name="s")


def _stream_spmm(xs_hbm, acc_sh, src_v, dst_v, gbuf0, gbuf1, sem0, sem1, nch):
    pltpu.async_copy(xs_hbm.at[src_v.at[0]], gbuf0, sem0)

    @pl.loop(0, nch // 2)
    def _(h):
        g0 = 2 * h
        pltpu.async_copy(xs_hbm.at[src_v.at[g0 + 1]], gbuf1, sem1)
        pltpu.make_async_copy(xs_hbm.at[src_v.at[g0]], gbuf0, sem0).wait()
        pltpu.sync_copy(gbuf0, acc_sh.at[dst_v.at[g0]], add=True)

        @pl.when(g0 + 2 < nch)
        def _():
            pltpu.async_copy(xs_hbm.at[src_v.at[g0 + 2]], gbuf0, sem0)

        pltpu.make_async_copy(xs_hbm.at[src_v.at[g0 + 1]], gbuf1, sem1).wait()
        pltpu.sync_copy(gbuf1, acc_sh.at[dst_v.at[g0 + 1]], add=True)


def _panel_sweep(xs_hbm, out_hbm, acc_sh, src_v, dst_v, gbuf0, gbuf1,
                 sem0, sem1, c, s):
    obase = c * HALF + s * RPH
    abase = s * RPH
    pltpu.sync_copy(xs_hbm.at[pl.ds(obase, RPH)], acc_sh.at[pl.ds(abase, RPH)])
    plsc.subcore_barrier()
    _stream_spmm(xs_hbm, acc_sh, src_v, dst_v, gbuf0, gbuf1, sem0, sem1, CPT)
    plsc.subcore_barrier()
    pltpu.sync_copy(acc_sh.at[pl.ds(abase, RPH)], out_hbm.at[pl.ds(obase, RPH)])
    plsc.subcore_barrier()


def _deg_body(dl0_hbm, dl1_hbm, deg_hbm, dst_v, ones_v, z_v, acc_sh, sem0):
    c = lax.axis_index("c")
    s = lax.axis_index("s")

    @pl.loop(0, CH)
    def _(i):
        for j in range(128 // 16):
            ones_v[i, pl.ds(j * 16, 16)] = jnp.ones((16,), jnp.float32)
            z_v[i, pl.ds(j * 16, 16)] = jnp.zeros((16,), jnp.float32)

    abase = s * RPH
    for r in range(RPH // CH):
        pltpu.sync_copy(z_v, acc_sh.at[pl.ds(abase + r * CH, CH)])
    rem = RPH % CH
    if rem:
        pltpu.sync_copy(z_v.at[pl.ds(0, rem)],
                        acc_sh.at[pl.ds(abase + RPH - rem, rem)])
    pltpu.sync_copy(z_v.at[pl.ds(0, 1)], acc_sh.at[pl.ds(HALF + s, 1)])

    cb = s * CPT

    @pl.when(c == 0)
    def _():
        pltpu.sync_copy(dl0_hbm.at[pl.ds(cb, CPT)], dst_v)

    @pl.when(c == 1)
    def _():
        pltpu.sync_copy(dl1_hbm.at[pl.ds(cb, CPT)], dst_v)

    plsc.subcore_barrier()

    @pl.loop(0, CPT)
    def _(g):
        pltpu.sync_copy(ones_v, acc_sh.at[dst_v.at[g]], add=True)

    plsc.subcore_barrier()
    pltpu.sync_copy(acc_sh.at[pl.ds(abase, RPH)],
                    deg_hbm.at[pl.ds(c * HALF + abase, RPH)])


def _l1_body(src_hbm, dl0_hbm, dl1_hbm, xsa_hbm, xsb_hbm, outa_hbm, outb_hbm,
             src_v, dst_v, gbuf0, gbuf1, acc_sh, sem0, sem1):
    c = lax.axis_index("c")
    s = lax.axis_index("s")
    cb = s * CPT
    pltpu.sync_copy(src_hbm.at[pl.ds(cb, CPT)], src_v)

    def sweeps():
        _panel_sweep(xsa_hbm, outa_hbm, acc_sh, src_v, dst_v, gbuf0, gbuf1,
                     sem0, sem1, c, s)
        _panel_sweep(xsb_hbm, outb_hbm, acc_sh, src_v, dst_v, gbuf0, gbuf1,
                     sem0, sem1, c, s)

    @pl.when(c == 0)
    def _():
        pltpu.sync_copy(dl0_hbm.at[pl.ds(cb, CPT)], dst_v)
        sweeps()

    @pl.when(c == 1)
    def _():
        pltpu.sync_copy(dl1_hbm.at[pl.ds(cb, CPT)], dst_v)
        sweeps()


def _l2_body(src_hbm, dl0_hbm, dl1_hbm, xs_hbm, out_hbm,
             src_v, dst_v, gbuf0, gbuf1, acc_sh, sem0, sem1):
    c = lax.axis_index("c")
    s = lax.axis_index("s")
    cb = s * CPT
    pltpu.sync_copy(src_hbm.at[pl.ds(cb, CPT)], src_v)

    @pl.when(c == 0)
    def _():
        pltpu.sync_copy(dl0_hbm.at[pl.ds(cb, CPT)], dst_v)

    @pl.when(c == 1)
    def _():
        pltpu.sync_copy(dl1_hbm.at[pl.ds(cb, CPT)], dst_v)

    _panel_sweep(xs_hbm, out_hbm, acc_sh, src_v, dst_v, gbuf0, gbuf1,
                 sem0, sem1, c, s)


_deg_call = functools.partial(
    pl.kernel, _deg_body,
    out_type=jax.ShapeDtypeStruct((NP, 128), jnp.float32),
    mesh=plsc.VectorSubcoreMesh(**_MESH),
    scratch_types=[
        pltpu.VMEM((CPT, CH), jnp.int32),
        pltpu.VMEM((CH, 128), jnp.float32),
        pltpu.VMEM((CH, 128), jnp.float32),
        pltpu.VMEM_SHARED((NPH, 128), jnp.float32),
        pltpu.SemaphoreType.DMA,
    ],
)()

_spmm_scratch = [
    pltpu.VMEM((CPT, CH), jnp.int32),
    pltpu.VMEM((CPT, CH), jnp.int32),
    pltpu.VMEM((CH, 128), jnp.float32),
    pltpu.VMEM((CH, 128), jnp.float32),
    pltpu.VMEM_SHARED((NPH, 128), jnp.float32),
    pltpu.SemaphoreType.DMA,
    pltpu.SemaphoreType.DMA,
]

_hbm128 = jax.ShapeDtypeStruct((NP, 128), jnp.float32)

_l1_call = functools.partial(
    pl.kernel, _l1_body,
    out_type=(_hbm128, _hbm128),
    mesh=plsc.VectorSubcoreMesh(**_MESH),
    scratch_types=_spmm_scratch,
)()

_l2_call = functools.partial(
    pl.kernel, _l2_body,
    out_type=_hbm128,
    mesh=plsc.VectorSubcoreMesh(**_MESH),
    scratch_types=_spmm_scratch,
)()


def _remap_body(dst_ref, dl0_ref, dl1_ref):
    i = pl.program_id(0)
    d = dst_ref[...]
    trash = HALF + i
    dl0_ref[...] = jnp.where(d < HALF, d, trash)
    dl1_ref[...] = jnp.where(d >= HALF, d - HALF, trash)


_remap_call = pl.pallas_call(
    _remap_body,
    grid=(NTILE,),
    in_specs=[pl.BlockSpec((CPT, CH), lambda i: (i, 0))],
    out_specs=[pl.BlockSpec((CPT, CH), lambda i: (i, 0))] * 2,
    out_shape=(jax.ShapeDtypeStruct((NCH, CH), jnp.int32),) * 2,
)


def _dinv(deg_ref):
    return lax.rsqrt(deg_ref[:, :1] + 1.0)


def _tc1_body(x_ref, deg_ref, wa_ref, wb_ref, oa_ref, ob_ref):
    dinv = _dinv(deg_ref)
    oa_ref[...] = dinv * jnp.dot(x_ref[...], wa_ref[...],
                                 preferred_element_type=jnp.float32)
    ob_ref[...] = dinv * jnp.dot(x_ref[...], wb_ref[...],
                                 preferred_element_type=jnp.float32)


def _tc2_body(aa_ref, ab_ref, deg_ref, b1a_ref, b1b_ref,
              w2a_ref, w2b_ref, o_ref):
    dinv = _dinv(deg_ref)
    h1a = jnp.maximum(dinv * aa_ref[...] + b1a_ref[...], 0.0)
    h1b = jnp.maximum(dinv * ab_ref[...] + b1b_ref[...], 0.0)
    o_ref[...] = dinv * (
        jnp.dot(h1a, w2a_ref[...], preferred_element_type=jnp.float32)
        + jnp.dot(h1b, w2b_ref[...], preferred_element_type=jnp.float32))


def _tc3_body(a_ref, deg_ref, b2_ref, o_ref):
    dinv = _dinv(deg_ref)
    o_ref[...] = jnp.maximum(dinv * a_ref[...] + b2_ref[...], 0.0)


_RB = 512
_GRID = (NP // _RB,)
_row = pl.BlockSpec((_RB, 128), lambda i: (i, 0))


def _const(shape):
    return pl.BlockSpec(shape, lambda i: tuple(0 for _ in shape))


_tc1_call = pl.pallas_call(
    _tc1_body,
    grid=_GRID,
    in_specs=[_row, _row, _const((128, 128)), _const((128, 128))],
    out_specs=[_row] * 2,
    out_shape=(_hbm128,) * 2,
)

_tc2_call = pl.pallas_call(
    _tc2_body,
    grid=_GRID,
    in_specs=[_row, _row, _row, _const((1, 128)), _const((1, 128)),
              _const((128, 128)), _const((128, 128))],
    out_specs=_row,
    out_shape=_hbm128,
)

_tc3_call = pl.pallas_call(
    _tc3_body,
    grid=_GRID,
    in_specs=[_row, _row, _const((1, 128))],
    out_specs=_row,
    out_shape=_hbm128,
)


def kernel(x, edge_index, idx, W1, b1, W2, b2):
    del idx
    src = edge_index[0].astype(jnp.int32)
    dst = edge_index[1].astype(jnp.int32)
    pad = jnp.full((EP - E,), N, dtype=jnp.int32)
    src2d = jnp.concatenate([src, pad]).reshape(NCH, CH)
    dst2d = jnp.concatenate([dst, pad]).reshape(NCH, CH)

    xpad = jnp.zeros((NP, 128), jnp.float32).at[:N].set(x)

    dl0, dl1 = _remap_call(dst2d)
    deg = _deg_call(dl0, dl1)

    xs1a, xs1b = _tc1_call(xpad, deg, W1[:, :128], W1[:, 128:])
    acc1a, acc1b = _l1_call(src2d, dl0, dl1, xs1a, xs1b)
    xs2 = _tc2_call(acc1a, acc1b, deg,
                    b1[:128].reshape(1, 128), b1[128:].reshape(1, 128),
                    W2[:128], W2[128:])
    acc2 = _l2_call(src2d, dl0, dl1, xs2)
    h = _tc3_call(acc2, deg, b2.reshape(1, 128))
    return h[:N]


# speedup vs baseline: 16.1641x; 3.5363x over previous
import functools

import jax
import jax.numpy as jnp
from jax import lax
from jax.experimental import pallas as pl
from jax.experimental.pallas import tpu as pltpu
from jax.experimental.pallas import tpu_sc as plsc

N = 10000
NP = 10240
E = 320000
CH = 128
NCH = 2560
EP = NCH * CH
NTILE = 16
RPT = NP // NTILE
HALF = NP // 2
NTR = NTILE
NPH = HALF + NTR
RPH = HALF // NTILE
CPT = NCH // NTILE

_MESH = dict(core_axis_name="c", subcore_axis_name="s")


def _gidx(idx_ref):
    return plsc.Indices(idx_ref, ignored_value=-1)


def _stream_spmm(xs_hbm, acc_sh, src_v, dst_v, gbuf0, gbuf1, sem0, sem1, nch):
    pltpu.async_copy(xs_hbm.at[_gidx(src_v.at[0])], gbuf0, sem0)

    @pl.loop(0, nch // 2)
    def _(h):
        g0 = 2 * h
        pltpu.async_copy(xs_hbm.at[_gidx(src_v.at[g0 + 1])], gbuf1, sem1)
        pltpu.make_async_copy(xs_hbm.at[_gidx(src_v.at[g0])], gbuf0, sem0).wait()
        pltpu.sync_copy(gbuf0, acc_sh.at[_gidx(dst_v.at[g0])], add=True)

        @pl.when(g0 + 2 < nch)
        def _():
            pltpu.async_copy(xs_hbm.at[_gidx(src_v.at[g0 + 2])], gbuf0, sem0)

        pltpu.make_async_copy(xs_hbm.at[_gidx(src_v.at[g0 + 1])], gbuf1, sem1).wait()
        pltpu.sync_copy(gbuf1, acc_sh.at[_gidx(dst_v.at[g0 + 1])], add=True)


def _panel_sweep(xs_hbm, out_hbm, acc_sh, src_v, dst_v, gbuf0, gbuf1,
                 sem0, sem1, c, s):
    obase = c * HALF + s * RPH
    abase = s * RPH
    pltpu.sync_copy(xs_hbm.at[pl.ds(obase, RPH)], acc_sh.at[pl.ds(abase, RPH)])
    plsc.subcore_barrier()
    _stream_spmm(xs_hbm, acc_sh, src_v, dst_v, gbuf0, gbuf1, sem0, sem1, CPT)
    plsc.subcore_barrier()
    pltpu.sync_copy(acc_sh.at[pl.ds(abase, RPH)], out_hbm.at[pl.ds(obase, RPH)])
    plsc.subcore_barrier()


def _deg_body(dl0_hbm, dl1_hbm, deg_hbm, dst_v, ones_v, z_v, acc_sh, sem0):
    c = lax.axis_index("c")
    s = lax.axis_index("s")

    @pl.loop(0, CH)
    def _(i):
        for j in range(128 // 16):
            ones_v[i, pl.ds(j * 16, 16)] = jnp.ones((16,), jnp.float32)
            z_v[i, pl.ds(j * 16, 16)] = jnp.zeros((16,), jnp.float32)

    abase = s * RPH
    for r in range(RPH // CH):
        pltpu.sync_copy(z_v, acc_sh.at[pl.ds(abase + r * CH, CH)])
    rem = RPH % CH
    if rem:
        pltpu.sync_copy(z_v.at[pl.ds(0, rem)],
                        acc_sh.at[pl.ds(abase + RPH - rem, rem)])
    cb = s * CPT

    @pl.when(c == 0)
    def _():
        pltpu.sync_copy(dl0_hbm.at[pl.ds(cb, CPT)], dst_v)

    @pl.when(c == 1)
    def _():
        pltpu.sync_copy(dl1_hbm.at[pl.ds(cb, CPT)], dst_v)

    plsc.subcore_barrier()

    @pl.loop(0, CPT)
    def _(g):
        pltpu.sync_copy(ones_v, acc_sh.at[_gidx(dst_v.at[g])], add=True)

    plsc.subcore_barrier()
    pltpu.sync_copy(acc_sh.at[pl.ds(abase, RPH)],
                    deg_hbm.at[pl.ds(c * HALF + abase, RPH)])


def _l1_body(sl0_hbm, sl1_hbm, dl0_hbm, dl1_hbm, xsa_hbm, xsb_hbm,
             outa_hbm, outb_hbm,
             src_v, dst_v, gbuf0, gbuf1, acc_sh, sem0, sem1):
    c = lax.axis_index("c")
    s = lax.axis_index("s")
    cb = s * CPT

    def sweeps():
        _panel_sweep(xsa_hbm, outa_hbm, acc_sh, src_v, dst_v, gbuf0, gbuf1,
                     sem0, sem1, c, s)
        _panel_sweep(xsb_hbm, outb_hbm, acc_sh, src_v, dst_v, gbuf0, gbuf1,
                     sem0, sem1, c, s)

    @pl.when(c == 0)
    def _():
        pltpu.sync_copy(sl0_hbm.at[pl.ds(cb, CPT)], src_v)
        pltpu.sync_copy(dl0_hbm.at[pl.ds(cb, CPT)], dst_v)
        sweeps()

    @pl.when(c == 1)
    def _():
        pltpu.sync_copy(sl1_hbm.at[pl.ds(cb, CPT)], src_v)
        pltpu.sync_copy(dl1_hbm.at[pl.ds(cb, CPT)], dst_v)
        sweeps()


def _l2_body(sl0_hbm, sl1_hbm, dl0_hbm, dl1_hbm, xs_hbm, out_hbm,
             src_v, dst_v, gbuf0, gbuf1, acc_sh, sem0, sem1):
    c = lax.axis_index("c")
    s = lax.axis_index("s")
    cb = s * CPT

    @pl.when(c == 0)
    def _():
        pltpu.sync_copy(sl0_hbm.at[pl.ds(cb, CPT)], src_v)
        pltpu.sync_copy(dl0_hbm.at[pl.ds(cb, CPT)], dst_v)

    @pl.when(c == 1)
    def _():
        pltpu.sync_copy(sl1_hbm.at[pl.ds(cb, CPT)], src_v)
        pltpu.sync_copy(dl1_hbm.at[pl.ds(cb, CPT)], dst_v)

    _panel_sweep(xs_hbm, out_hbm, acc_sh, src_v, dst_v, gbuf0, gbuf1,
                 sem0, sem1, c, s)


_deg_call = functools.partial(
    pl.kernel, _deg_body,
    out_type=jax.ShapeDtypeStruct((NP, 128), jnp.float32),
    mesh=plsc.VectorSubcoreMesh(**_MESH),
    scratch_types=[
        pltpu.VMEM((CPT, CH), jnp.int32),
        pltpu.VMEM((CH, 128), jnp.float32),
        pltpu.VMEM((CH, 128), jnp.float32),
        pltpu.VMEM_SHARED((NPH, 128), jnp.float32),
        pltpu.SemaphoreType.DMA,
    ],
)()

_spmm_scratch = [
    pltpu.VMEM((CPT, CH), jnp.int32),
    pltpu.VMEM((CPT, CH), jnp.int32),
    pltpu.VMEM((CH, 128), jnp.float32),
    pltpu.VMEM((CH, 128), jnp.float32),
    pltpu.VMEM_SHARED((NPH, 128), jnp.float32),
    pltpu.SemaphoreType.DMA,
    pltpu.SemaphoreType.DMA,
]

_hbm128 = jax.ShapeDtypeStruct((NP, 128), jnp.float32)

_l1_call = functools.partial(
    pl.kernel, _l1_body,
    out_type=(_hbm128, _hbm128),
    mesh=plsc.VectorSubcoreMesh(**_MESH),
    scratch_types=_spmm_scratch,
)()

_l2_call = functools.partial(
    pl.kernel, _l2_body,
    out_type=_hbm128,
    mesh=plsc.VectorSubcoreMesh(**_MESH),
    scratch_types=_spmm_scratch,
)()


def _remap_body(src_ref, dst_ref, sl0_ref, sl1_ref, dl0_ref, dl1_ref):
    sr = src_ref[...]
    d = dst_ref[...]
    m0 = d < HALF
    m1 = jnp.logical_and(d >= HALF, d < N)
    neg = jnp.full_like(d, -1)
    sl0_ref[...] = jnp.where(m0, sr, neg)
    sl1_ref[...] = jnp.where(m1, sr, neg)
    dl0_ref[...] = jnp.where(m0, d, neg)
    dl1_ref[...] = jnp.where(m1, d - HALF, neg)


_remap_call = pl.pallas_call(
    _remap_body,
    grid=(NTILE,),
    in_specs=[pl.BlockSpec((CPT, CH), lambda i: (i, 0))] * 2,
    out_specs=[pl.BlockSpec((CPT, CH), lambda i: (i, 0))] * 4,
    out_shape=(jax.ShapeDtypeStruct((NCH, CH), jnp.int32),) * 4,
)


def _dinv(deg_ref):
    return lax.rsqrt(deg_ref[:, :1] + 1.0)


def _tc1_body(x_ref, deg_ref, wa_ref, wb_ref, oa_ref, ob_ref):
    dinv = _dinv(deg_ref)
    oa_ref[...] = dinv * jnp.dot(x_ref[...], wa_ref[...],
                                 preferred_element_type=jnp.float32)
    ob_ref[...] = dinv * jnp.dot(x_ref[...], wb_ref[...],
                                 preferred_element_type=jnp.float32)


def _tc2_body(aa_ref, ab_ref, deg_ref, b1a_ref, b1b_ref,
              w2a_ref, w2b_ref, o_ref):
    dinv = _dinv(deg_ref)
    h1a = jnp.maximum(dinv * aa_ref[...] + b1a_ref[...], 0.0)
    h1b = jnp.maximum(dinv * ab_ref[...] + b1b_ref[...], 0.0)
    o_ref[...] = dinv * (
        jnp.dot(h1a, w2a_ref[...], preferred_element_type=jnp.float32)
        + jnp.dot(h1b, w2b_ref[...], preferred_element_type=jnp.float32))


def _tc3_body(a_ref, deg_ref, b2_ref, o_ref):
    dinv = _dinv(deg_ref)
    o_ref[...] = jnp.maximum(dinv * a_ref[...] + b2_ref[...], 0.0)


_RB = 512
_GRID = (NP // _RB,)
_row = pl.BlockSpec((_RB, 128), lambda i: (i, 0))


def _const(shape):
    return pl.BlockSpec(shape, lambda i: tuple(0 for _ in shape))


_tc1_call = pl.pallas_call(
    _tc1_body,
    grid=_GRID,
    in_specs=[_row, _row, _const((128, 128)), _const((128, 128))],
    out_specs=[_row] * 2,
    out_shape=(_hbm128,) * 2,
)

_tc2_call = pl.pallas_call(
    _tc2_body,
    grid=_GRID,
    in_specs=[_row, _row, _row, _const((1, 128)), _const((1, 128)),
              _const((128, 128)), _const((128, 128))],
    out_specs=_row,
    out_shape=_hbm128,
)

_tc3_call = pl.pallas_call(
    _tc3_body,
    grid=_GRID,
    in_specs=[_row, _row, _const((1, 128))],
    out_specs=_row,
    out_shape=_hbm128,
)


def kernel(x, edge_index, idx, W1, b1, W2, b2):
    del idx
    src = edge_index[0].astype(jnp.int32)
    dst = edge_index[1].astype(jnp.int32)
    pad = jnp.full((EP - E,), N, dtype=jnp.int32)
    src2d = jnp.concatenate([src, pad]).reshape(NCH, CH)
    dst2d = jnp.concatenate([dst, pad]).reshape(NCH, CH)

    xpad = jnp.zeros((NP, 128), jnp.float32).at[:N].set(x)

    sl0, sl1, dl0, dl1 = _remap_call(src2d, dst2d)
    deg = _deg_call(dl0, dl1)

    xs1a, xs1b = _tc1_call(xpad, deg, W1[:, :128], W1[:, 128:])
    acc1a, acc1b = _l1_call(sl0, sl1, dl0, dl1, xs1a, xs1b)
    xs2 = _tc2_call(acc1a, acc1b, deg,
                    b1[:128].reshape(1, 128), b1[128:].reshape(1, 128),
                    W2[:128], W2[128:])
    acc2 = _l2_call(sl0, sl1, dl0, dl1, xs2)
    h = _tc3_call(acc2, deg, b2.reshape(1, 128))
    return h[:N]
